# baseline (speedup 1.0000x reference)
import jax
import jax.numpy as jnp
from jax.experimental import pallas as pl
from jax.experimental.pallas import tpu as pltpu

_CFG = [(1024, 8, 1, 0, 48), (384, 12, 2, 48, 96), (128, 16, 2, 96, 192), (128, 16, 3, 192, 384)]


def _elu(x):
    return jnp.where(x > 0, x, jnp.exp(x) - 1.0)


def _xconv_layer(p, feat, pr, m, K, d, cin, cout, mb, BB):
    B, n, _ = p.shape
    cd = cout // 4
    C = cd + cin
    R = BB * mb
    has_feat = cin > 0
    ctr = p[:, :m]
    pT = jnp.transpose(p, (0, 2, 1))
    pf = p if cin == 0 else jnp.concatenate([p, feat], axis=2)
    C0 = 3 + cin
    W1, b1 = pr['W1'], pr['b1'].reshape(1, cd)
    W2, b2 = pr['W2'], pr['b2'].reshape(1, cd)
    Wt, bt = pr['Wt'], pr['bt'].reshape(1, K * K)
    Wc, bc = pr['Wc'], pr['bc'].reshape(1, cout)

    def kern(*refs):
        it = iter(refs)
        ctr_ref = next(it)
        pT_ref = next(it)
        pf_ref = next(it)
        W1_ref, b1_ref, W2_ref, b2_ref = next(it), next(it), next(it), next(it)
        Wt_ref, bt_ref, Wc_ref, bc_ref = next(it), next(it), next(it), next(it)
        out_ref = next(it)

        prow = [pT_ref[:, c:c + 1, :] for c in range(3)]
        ccol3 = [ctr_ref[:, :, c:c + 1] for c in range(3)]
        d2 = None
        for c in range(3):
            diff = ccol3[c] - prow[c]
            d2 = diff * diff if d2 is None else d2 + diff * diff
        irow = jax.lax.broadcasted_iota(jnp.int32, (BB, mb, n), 1)
        icol = jax.lax.broadcasted_iota(jnp.int32, (BB, mb, n), 2)
        d2 = jnp.where(irow == icol, jnp.float32(jnp.inf), d2)
        d2 = d2.reshape(R, n)
        ccol = [x.reshape(R, 1) for x in ccol3]
        iota = jax.lax.broadcasted_iota(jnp.int32, (R, n), 1)
        pfb = pf_ref[...]
        rel = [None] * K
        fk = [None] * K
        if has_feat:
            fk[0] = pfb[:, :mb, 3:].reshape(R, cin)
        for t in range(1, K * d):
            am = jnp.argmin(d2, axis=1).astype(jnp.int32)[:, None]
            oh = iota == am
            if t % d == 0:
                k = t // d
                ohf = oh.astype(jnp.float32)
                if BB == 1:
                    g = jnp.dot(ohf, pfb[0], preferred_element_type=jnp.float32)
                else:
                    oh3 = ohf.reshape(BB, mb, n)
                    g = jnp.concatenate(
                        [jnp.dot(oh3[bb], pfb[bb],
                                 preferred_element_type=jnp.float32)
                         for bb in range(BB)], axis=0)
                rel[k] = [g[:, c:c + 1] - ccol[c] for c in range(3)]
                if has_feat:
                    fk[k] = g[:, 3:3 + cin]
            if t + 1 < K * d:
                d2 = jnp.where(oh, jnp.float32(jnp.inf), d2)

        h = [None] * K
        h1_0 = _elu(b1_ref[0:1, :])
        h2_0 = _elu(jnp.dot(h1_0, W2_ref[:, :],
                            preferred_element_type=jnp.float32) + b2_ref[0:1, :])
        h[0] = (jnp.concatenate([jnp.broadcast_to(h2_0, (R, cd)), fk[0]], axis=1)
                if has_feat else jnp.broadcast_to(h2_0, (R, cd)))
        for k in range(1, K):
            h1 = b1_ref[0:1, :]
            for c in range(3):
                h1 = h1 + rel[k][c] * W1_ref[c:c + 1, :]
            h1 = _elu(h1)
            h2 = _elu(jnp.dot(h1, W2_ref[:, :],
                              preferred_element_type=jnp.float32) + b2_ref[0:1, :])
            h[k] = jnp.concatenate([h2, fk[k]], axis=1) if has_feat else h2

        T = jnp.broadcast_to(bt_ref[0:1, :], (R, K * K))
        for k in range(1, K):
            for c in range(3):
                T = T + rel[k][c] * Wt_ref[3 * k + c:3 * k + c + 1, :]

        acc = jnp.broadcast_to(bc_ref[0:1, :], (R, cout))
        for k in range(K):
            hT = None
            for j in range(K):
                term = T[:, k * K + j:k * K + j + 1] * h[j]
                hT = term if hT is None else hT + term
            acc = acc + jnp.dot(hT, Wc_ref[k * C:(k + 1) * C, :],
                                preferred_element_type=jnp.float32)
        out_ref[...] = _elu(acc).reshape(BB, mb, cout)

    in_specs = [
        pl.BlockSpec((BB, mb, 3), lambda b, i: (b, i, 0)),
        pl.BlockSpec((BB, 3, n), lambda b, i: (b, 0, 0)),
        pl.BlockSpec((BB, n, C0), lambda b, i: (b, 0, 0)),
    ]
    operands = [ctr, pT, pf]
    for w in (W1, b1, W2, b2, Wt, bt, Wc, bc):
        in_specs.append(pl.BlockSpec(w.shape, lambda b, i: (0, 0)))
        operands.append(w)
    out = pl.pallas_call(
        kern,
        grid=(B // BB, m // mb),
        in_specs=in_specs,
        out_specs=pl.BlockSpec((BB, mb, cout), lambda b, i: (b, i, 0)),
        out_shape=jax.ShapeDtypeStruct((B, m, cout), jnp.float32),
        compiler_params=pltpu.CompilerParams(
            dimension_semantics=("parallel", "parallel")),
    )(*operands)
    return ctr, out


def _head(x, fc, B):
    R = x.shape[0]
    npts = R // B
    nc = fc['W3'].shape[1]

    def kern(x_ref, W1_ref, b1_ref, g1_ref, be1_ref, W2_ref, b2_ref,
             g2_ref, be2_ref, W3_ref, b3_ref, out_ref):
        def bn(y, g, b):
            mu = jnp.mean(y, axis=0, keepdims=True)
            v = jnp.mean((y - mu) ** 2, axis=0, keepdims=True)
            return g * (y - mu) / jnp.sqrt(v + 1e-5) + b

        y = _elu(jnp.dot(x_ref[...], W1_ref[...],
                         preferred_element_type=jnp.float32) + b1_ref[0:1, :])
        y = bn(y, g1_ref[0:1, :], be1_ref[0:1, :])
        y = _elu(jnp.dot(y, W2_ref[...],
                         preferred_element_type=jnp.float32) + b2_ref[0:1, :])
        y = bn(y, g2_ref[0:1, :], be2_ref[0:1, :])
        logits = jnp.dot(y, W3_ref[...],
                         preferred_element_type=jnp.float32) + b3_ref[0:1, :]
        row = jax.lax.broadcasted_iota(jnp.int32, (B, R), 1)
        grp = jax.lax.broadcasted_iota(jnp.int32, (B, R), 0)
        S = jnp.where(row // npts == grp, jnp.float32(1.0 / npts), 0.0)
        z = jnp.dot(S, logits, preferred_element_type=jnp.float32)
        z = z - jnp.max(z, axis=1, keepdims=True)
        out_ref[...] = z - jnp.log(jnp.sum(jnp.exp(z), axis=1, keepdims=True))

    args = (x, fc['W1'], fc['b1'].reshape(1, -1), fc['g1'].reshape(1, -1),
            fc['be1'].reshape(1, -1), fc['W2'], fc['b2'].reshape(1, -1),
            fc['g2'].reshape(1, -1), fc['be2'].reshape(1, -1),
            fc['W3'], fc['b3'].reshape(1, -1))
    return pl.pallas_call(
        kern,
        out_shape=jax.ShapeDtypeStruct((B, nc), jnp.float32),
    )(*args)


def kernel(pos, params):
    p = pos
    feat = None
    mbs = [1024, 384, 128, 128]
    bbs = [2, 2, 4, 2]
    for i, (m, K, d, cin, cout) in enumerate(_CFG):
        p, feat = _xconv_layer(p, feat, params['l%d' % i], m, K, d, cin, cout,
                               mbs[i], bbs[i])
    B = feat.shape[0]
    x = feat.reshape(-1, feat.shape[-1])
    return _head(x, params['fc'], B)


# speedup vs baseline: 1.1271x; 1.1271x over previous
import jax
import jax.numpy as jnp
from jax.experimental import pallas as pl
from jax.experimental.pallas import tpu as pltpu

_CFG = [(1024, 8, 1, 0, 48), (384, 12, 2, 48, 96), (128, 16, 2, 96, 192), (128, 16, 3, 192, 384)]


def _elu(x):
    return jnp.where(x > 0, x, jnp.exp(x) - 1.0)


def _xconv_layer(p, feat, pr, m, K, d, cin, cout, mb, BB):
    B, n, _ = p.shape
    cd = cout // 4
    C = cd + cin
    R = BB * mb
    has_feat = cin > 0
    ctr = p[:, :m]
    pT = jnp.transpose(p, (0, 2, 1))
    pf = p if cin == 0 else jnp.concatenate([p, feat], axis=2)
    C0 = 3 + cin
    W1, b1 = pr['W1'], pr['b1'].reshape(1, cd)
    W2, b2 = pr['W2'], pr['b2'].reshape(1, cd)
    Wt, bt = pr['Wt'], pr['bt'].reshape(1, K * K)
    Wc, bc = pr['Wc'], pr['bc'].reshape(1, cout)

    def kern(*refs):
        it = iter(refs)
        ctr_ref = next(it)
        pT_ref = next(it)
        pf_ref = next(it)
        W1_ref, b1_ref, W2_ref, b2_ref = next(it), next(it), next(it), next(it)
        Wt_ref, bt_ref, Wc_ref, bc_ref = next(it), next(it), next(it), next(it)
        out_ref = next(it)

        prow = [pT_ref[:, c:c + 1, :] for c in range(3)]
        ccol3 = [ctr_ref[:, :, c:c + 1] for c in range(3)]
        d2 = None
        for c in range(3):
            diff = ccol3[c] - prow[c]
            d2 = diff * diff if d2 is None else d2 + diff * diff
        irow = jax.lax.broadcasted_iota(jnp.int32, (BB, mb, n), 1)
        icol = jax.lax.broadcasted_iota(jnp.int32, (BB, mb, n), 2)
        d2 = jnp.where(irow == icol, jnp.float32(jnp.inf), d2)
        d2 = d2.reshape(R, n)
        ccol = [x.reshape(R, 1) for x in ccol3]
        iota = jax.lax.broadcasted_iota(jnp.int32, (R, n), 1)
        pfb = pf_ref[...]
        rel = [None] * K
        fk = [None] * K
        if has_feat:
            fk[0] = pfb[:, :mb, 3:].reshape(R, cin)
        for t in range(1, K * d):
            am = jnp.argmin(d2, axis=1).astype(jnp.int32)[:, None]
            oh = iota == am
            if t % d == 0:
                k = t // d
                ohf = oh.astype(jnp.float32)
                if BB == 1:
                    g = jnp.dot(ohf, pfb[0], preferred_element_type=jnp.float32)
                else:
                    oh3 = ohf.reshape(BB, mb, n)
                    g = jnp.concatenate(
                        [jnp.dot(oh3[bb], pfb[bb],
                                 preferred_element_type=jnp.float32)
                         for bb in range(BB)], axis=0)
                rel[k] = [g[:, c:c + 1] - ccol[c] for c in range(3)]
                if has_feat:
                    fk[k] = g[:, 3:3 + cin]
            if t + 1 < K * d:
                d2 = jnp.where(oh, jnp.float32(jnp.inf), d2)

        h = [None] * K
        h1_0 = _elu(b1_ref[0:1, :])
        h2_0 = _elu(jnp.dot(h1_0, W2_ref[:, :],
                            preferred_element_type=jnp.float32) + b2_ref[0:1, :])
        h[0] = (jnp.concatenate([jnp.broadcast_to(h2_0, (R, cd)), fk[0]], axis=1)
                if has_feat else jnp.broadcast_to(h2_0, (R, cd)))
        for k in range(1, K):
            h1 = b1_ref[0:1, :]
            for c in range(3):
                h1 = h1 + rel[k][c] * W1_ref[c:c + 1, :]
            h1 = _elu(h1)
            h2 = _elu(jnp.dot(h1, W2_ref[:, :],
                              preferred_element_type=jnp.float32) + b2_ref[0:1, :])
            h[k] = jnp.concatenate([h2, fk[k]], axis=1) if has_feat else h2

        T = jnp.broadcast_to(bt_ref[0:1, :], (R, K * K))
        for k in range(1, K):
            for c in range(3):
                T = T + rel[k][c] * Wt_ref[3 * k + c:3 * k + c + 1, :]

        acc = jnp.broadcast_to(bc_ref[0:1, :], (R, cout))
        for k in range(K):
            hT = None
            for j in range(K):
                term = T[:, k * K + j:k * K + j + 1] * h[j]
                hT = term if hT is None else hT + term
            acc = acc + jnp.dot(hT, Wc_ref[k * C:(k + 1) * C, :],
                                preferred_element_type=jnp.float32)
        out_ref[...] = _elu(acc).reshape(BB, mb, cout)

    in_specs = [
        pl.BlockSpec((BB, mb, 3), lambda b, i: (b, i, 0)),
        pl.BlockSpec((BB, 3, n), lambda b, i: (b, 0, 0)),
        pl.BlockSpec((BB, n, C0), lambda b, i: (b, 0, 0)),
    ]
    operands = [ctr, pT, pf]
    for w in (W1, b1, W2, b2, Wt, bt, Wc, bc):
        in_specs.append(pl.BlockSpec(w.shape, lambda b, i: (0, 0)))
        operands.append(w)
    out = pl.pallas_call(
        kern,
        grid=(B // BB, m // mb),
        in_specs=in_specs,
        out_specs=pl.BlockSpec((BB, mb, cout), lambda b, i: (b, i, 0)),
        out_shape=jax.ShapeDtypeStruct((B, m, cout), jnp.float32),
        compiler_params=pltpu.CompilerParams(
            dimension_semantics=("parallel", "parallel")),
    )(*operands)
    return ctr, out


def _head(x, fc, B):
    R = x.shape[0]
    npts = R // B
    nc = fc['W3'].shape[1]

    def kern(x_ref, W1_ref, b1_ref, g1_ref, be1_ref, W2_ref, b2_ref,
             g2_ref, be2_ref, W3_ref, b3_ref, out_ref):
        def bn(y, g, b):
            mu = jnp.mean(y, axis=0, keepdims=True)
            v = jnp.mean((y - mu) ** 2, axis=0, keepdims=True)
            return g * (y - mu) / jnp.sqrt(v + 1e-5) + b

        y = _elu(jnp.dot(x_ref[...], W1_ref[...],
                         preferred_element_type=jnp.float32) + b1_ref[0:1, :])
        y = bn(y, g1_ref[0:1, :], be1_ref[0:1, :])
        y = _elu(jnp.dot(y, W2_ref[...],
                         preferred_element_type=jnp.float32) + b2_ref[0:1, :])
        y = bn(y, g2_ref[0:1, :], be2_ref[0:1, :])
        logits = jnp.dot(y, W3_ref[...],
                         preferred_element_type=jnp.float32) + b3_ref[0:1, :]
        row = jax.lax.broadcasted_iota(jnp.int32, (B, R), 1)
        grp = jax.lax.broadcasted_iota(jnp.int32, (B, R), 0)
        S = jnp.where(row // npts == grp, jnp.float32(1.0 / npts), 0.0)
        z = jnp.dot(S, logits, preferred_element_type=jnp.float32)
        z = z - jnp.max(z, axis=1, keepdims=True)
        out_ref[...] = z - jnp.log(jnp.sum(jnp.exp(z), axis=1, keepdims=True))

    args = (x, fc['W1'], fc['b1'].reshape(1, -1), fc['g1'].reshape(1, -1),
            fc['be1'].reshape(1, -1), fc['W2'], fc['b2'].reshape(1, -1),
            fc['g2'].reshape(1, -1), fc['be2'].reshape(1, -1),
            fc['W3'], fc['b3'].reshape(1, -1))
    return pl.pallas_call(
        kern,
        out_shape=jax.ShapeDtypeStruct((B, nc), jnp.float32),
    )(*args)


def kernel(pos, params):
    p = pos
    feat = None
    mbs = [1024, 384, 128, 128]
    bbs = [1, 1, 4, 2]
    for i, (m, K, d, cin, cout) in enumerate(_CFG):
        p, feat = _xconv_layer(p, feat, params['l%d' % i], m, K, d, cin, cout,
                               mbs[i], bbs[i])
    B = feat.shape[0]
    x = feat.reshape(-1, feat.shape[-1])
    return _head(x, params['fc'], B)
